# baseline (speedup 1.0000x reference)
import functools

import jax
import jax.numpy as jnp
from jax import lax
from jax.experimental import pallas as pl
from jax.experimental.pallas import tpu as pltpu
from jax.experimental.pallas import tpu_sc as plsc

_NC = 2
_NS = 16
_CW = 128
_CH = 80
_NB = 4
_NPAD = 10080
_EPAD = 327680


_RBIG = 640
_RSUB = 32


def _row_chunks(n, s):
    nfull = pl.cdiv(n, _RBIG) - 1
    last = n - nfull * _RBIG
    nsub = jnp.where(s < nfull, _RBIG // _RSUB, last // _RSUB)
    return s * _RBIG, nsub


@functools.lru_cache(maxsize=None)
def _make_deg(n, e):
    np_, ep = _NPAD, _EPAD
    crows = ep // _CW
    tr = crows // (_NC * _NS)
    assert tr * _NC * _NS == crows and tr % 8 == 0, (n, e)
    infl = 8
    mesh = plsc.VectorSubcoreMesh(core_axis_name="c", subcore_axis_name="s")

    @functools.partial(
        pl.kernel,
        mesh=mesh,
        out_type=jax.ShapeDtypeStruct((_NC, np_, 16), jnp.float32),
        scratch_types=[
            pltpu.VMEM((tr, _CW), jnp.int32),
            pltpu.VMEM((_CW, 16), jnp.float32),
            pltpu.VMEM((_RSUB, 16), jnp.float32),
            pltpu.VMEM_SHARED((np_, 16), jnp.float32),
            pltpu.SemaphoreType.DMA,
        ],
    )
    def deg_kernel(dst2_hbm, out_hbm, idx2, ones_v, zbuf, acc, wsem):
        c = lax.axis_index("c")
        s = lax.axis_index("s")
        w = c * _NS + s

        pltpu.sync_copy(dst2_hbm.at[pl.ds(w * tr, tr)], idx2)

        def fill_ones(r, carry):
            ones_v[r, :] = jnp.ones((16,), jnp.float32)
            return carry

        lax.fori_loop(0, _CW, fill_ones, None)

        def fill_zero(r, carry):
            zbuf[r, :] = jnp.zeros((16,), jnp.float32)
            return carry

        lax.fori_loop(0, _RSUB, fill_zero, None)

        r0, nsub = _row_chunks(np_, s)

        def zero_acc(k, carry):
            pltpu.sync_copy(zbuf, acc.at[pl.ds(r0 + k * _RSUB, _RSUB)])
            return carry

        lax.fori_loop(0, nsub, zero_acc, None)
        plsc.subcore_barrier()

        def fire(j):
            pltpu.async_copy(ones_v, acc.at[idx2.at[j]], wsem, add=True)

        def drain_one():
            pltpu.make_async_copy(ones_v, acc.at[idx2.at[0]], wsem).wait()

        def prime(j, carry):
            fire(j)
            return carry

        lax.fori_loop(0, infl, prime, None)

        def steady(j, carry):
            drain_one()
            fire(j)
            return carry

        lax.fori_loop(infl, tr, steady, None)

        def drain(j, carry):
            drain_one()
            return carry

        lax.fori_loop(0, infl, drain, None)
        plsc.subcore_barrier()

        def wout(k, carry):
            rr = r0 + k * _RSUB
            pltpu.sync_copy(acc.at[pl.ds(rr, _RSUB)],
                            out_hbm.at[c, pl.ds(rr, _RSUB)])
            return carry

        lax.fori_loop(0, nsub, wout, None)

    return deg_kernel


@functools.lru_cache(maxsize=None)
def _make_agg(n, e, d):
    np_, ep = _NPAD, _EPAD
    crows = ep // _CW
    tr = crows // (_NC * _NS)
    assert tr * _NC * _NS == crows and tr % (_NB * 2) == 0 and tr % 8 == 0
    mesh = plsc.VectorSubcoreMesh(core_axis_name="c", subcore_axis_name="s")

    @functools.partial(
        pl.kernel,
        mesh=mesh,
        out_type=jax.ShapeDtypeStruct((_NC, np_, d), jnp.float32),
        scratch_types=[
            pltpu.VMEM((_CH,), jnp.int32),
            pltpu.VMEM((_CH,), jnp.int32),
            pltpu.VMEM((_CH,), jnp.int32),
            pltpu.VMEM((_CH,), jnp.int32),
            pltpu.VMEM((_CH, d), jnp.float32),
            pltpu.VMEM((_CH, d), jnp.float32),
            pltpu.VMEM((_RSUB, d), jnp.float32),
            pltpu.VMEM_SHARED((np_, d), jnp.float32),
            pltpu.SemaphoreType.DMA,
            pltpu.SemaphoreType.DMA,
            pltpu.SemaphoreType.DMA,
            pltpu.SemaphoreType.DMA,
        ],
    )
    def agg_kernel(h_hbm, src_hbm, dst_hbm, out_hbm,
                   isa, ida, isb, idb, rows_a, rows_b,
                   zbuf, acc, gsem_a, gsem_b, wsem_a, wsem_b):
        c = lax.axis_index("c")
        s = lax.axis_index("s")
        w = c * _NS + s
        ept = _EPAD // (_NC * _NS)
        base = w * ept

        def fill_zero(r, carry):
            def col(j, carry2):
                zbuf[r, pl.ds(j * 16, 16)] = jnp.zeros((16,), jnp.float32)
                return carry2
            return lax.fori_loop(0, d // 16, col, carry)

        lax.fori_loop(0, _RSUB, fill_zero, None)

        r0, nsub = _row_chunks(np_, s)

        def zero_acc(k, carry):
            pltpu.sync_copy(zbuf, acc.at[pl.ds(r0 + k * _RSUB, _RSUB)])
            return carry

        lax.fori_loop(0, nsub, zero_acc, None)
        plsc.subcore_barrier()

        def pair(k, carry):
            o0 = base + (2 * k) * _CH
            o1 = o0 + _CH
            pltpu.sync_copy(src_hbm.at[pl.ds(o0, _CH)], isa)
            pltpu.sync_copy(dst_hbm.at[pl.ds(o0, _CH)], ida)
            g0 = pltpu.async_copy(h_hbm.at[isa], rows_a, gsem_a)
            pltpu.sync_copy(src_hbm.at[pl.ds(o1, _CH)], isb)
            pltpu.sync_copy(dst_hbm.at[pl.ds(o1, _CH)], idb)
            g1 = pltpu.async_copy(h_hbm.at[isb], rows_b, gsem_b)
            g0.wait()
            w0 = pltpu.async_copy(rows_a, acc.at[ida], wsem_a, add=True)
            g1.wait()
            w1 = pltpu.async_copy(rows_b, acc.at[idb], wsem_b, add=True)
            w0.wait()
            w1.wait()
            return carry

        lax.fori_loop(0, ept // (2 * _CH), pair, None)
        plsc.subcore_barrier()

        def wout(k, carry):
            rr = r0 + k * _RSUB
            pltpu.sync_copy(acc.at[pl.ds(rr, _RSUB)],
                            out_hbm.at[c, pl.ds(rr, _RSUB)])
            return carry

        lax.fori_loop(0, nsub, wout, None)

    return agg_kernel


_BR = 1000


def _dinv_block(degp_ref):
    deg = degp_ref[0, :, 0:1] + degp_ref[1, :, 0:1] + 1.0
    return lax.rsqrt(deg)


def _tc1_body(x_ref, degp_ref, wp_ref, bp_ref, w1_ref, o_ref):
    dinv = _dinv_block(degp_ref)
    wc = jnp.dot(wp_ref[...], w1_ref[...], preferred_element_type=jnp.float32)
    bc = jnp.dot(bp_ref[...].reshape(1, -1), w1_ref[...],
                 preferred_element_type=jnp.float32)
    h = jnp.dot(x_ref[...], wc, preferred_element_type=jnp.float32) + bc
    o_ref[...] = h * dinv


def _tc2_body(a_ref, hs1_ref, degp_ref, b1_ref, w2_ref, o_ref):
    dinv = _dinv_block(degp_ref)
    t = (a_ref[0] + a_ref[1] + hs1_ref[...]) * dinv + b1_ref[...]
    t = jnp.maximum(t, 0.0)
    o_ref[...] = jnp.dot(t, w2_ref[...],
                         preferred_element_type=jnp.float32) * dinv


def _tc3_body(a_ref, hs2_ref, degp_ref, b2_ref, o_ref):
    dinv = _dinv_block(degp_ref)
    y = (a_ref[0] + a_ref[1] + hs2_ref[...]) * dinv + b2_ref[...]
    nrm = jnp.sqrt(jnp.sum(y * y, axis=1, keepdims=True))
    o_ref[...] = y / jnp.maximum(nrm, 1e-12)


def _row_spec(d):
    return pl.BlockSpec((_BR, d), lambda i: (i, 0))


def _part_spec(d):
    return pl.BlockSpec((_NC, _BR, d), lambda i: (0, i, 0))


def _full_spec(shape):
    nd = len(shape)
    return pl.BlockSpec(shape, lambda i: (0,) * nd)


@functools.lru_cache(maxsize=None)
def _make_tc1(n, d):
    return pl.pallas_call(
        _tc1_body,
        grid=(n // _BR,),
        in_specs=[_row_spec(d), _part_spec(16), _full_spec((d, d)),
                  _full_spec((d,)), _full_spec((d, d))],
        out_specs=_row_spec(d),
        out_shape=jax.ShapeDtypeStruct((n, d), jnp.float32),
    )


@functools.lru_cache(maxsize=None)
def _make_tc2(n, d):
    return pl.pallas_call(
        _tc2_body,
        grid=(n // _BR,),
        in_specs=[_part_spec(d), _row_spec(d), _part_spec(16),
                  _full_spec((d,)), _full_spec((d, d))],
        out_specs=_row_spec(d),
        out_shape=jax.ShapeDtypeStruct((n, d), jnp.float32),
    )


@functools.lru_cache(maxsize=None)
def _make_tc3(n, d):
    return pl.pallas_call(
        _tc3_body,
        grid=(n // _BR,),
        in_specs=[_part_spec(d), _row_spec(d), _part_spec(16),
                  _full_spec((d,))],
        out_specs=_row_spec(d),
        out_shape=jax.ShapeDtypeStruct((n, d), jnp.float32),
    )


def kernel(x, edge_index, W_pre, b_pre, W1, b1, W2, b2):
    n, d = x.shape
    e = edge_index.shape[1]
    ei = edge_index.astype(jnp.int32)
    pad = _EPAD - e
    src_p = jnp.concatenate([ei[0], jnp.zeros((pad,), jnp.int32)])
    dst_p = jnp.concatenate([ei[1], jnp.full((pad,), n, jnp.int32)])
    src2 = src_p.reshape(_EPAD // _CW, _CW)
    dst2 = dst_p.reshape(_EPAD // _CW, _CW)

    degp = _make_deg(n, e)(dst2)
    hs1 = _make_tc1(n, d)(x, degp, W_pre, b_pre, W1)
    agg1 = _make_agg(n, e, d)(hs1, src_p, dst_p)
    hs2 = _make_tc2(n, d)(agg1, hs1, degp, b1, W2)
    agg2 = _make_agg(n, e, d)(hs2, src_p, dst_p)
    return _make_tc3(n, d)(agg2, hs2, degp, b2)


# speedup vs baseline: 3.1613x; 3.1613x over previous
import functools

import jax
import jax.numpy as jnp
from jax import lax
from jax.experimental import pallas as pl
from jax.experimental.pallas import tpu as pltpu
from jax.experimental.pallas import tpu_sc as plsc

_NC = 2
_NS = 16
_CW = 125
_NPAD = 10080


_RBIG = 640
_RSUB = 32


def _row_chunks(n, s):
    nfull = pl.cdiv(n, _RBIG) - 1
    last = n - nfull * _RBIG
    nsub = jnp.where(s < nfull, _RBIG // _RSUB, last // _RSUB)
    return s * _RBIG, nsub


@functools.lru_cache(maxsize=None)
def _make_deg(n, e):
    np_ = _NPAD
    crows = e // _CW
    tr = crows // (_NC * _NS)
    assert tr * _NC * _NS == crows and tr % 8 == 0, (n, e)
    infl = 8
    mesh = plsc.VectorSubcoreMesh(core_axis_name="c", subcore_axis_name="s")

    @functools.partial(
        pl.kernel,
        mesh=mesh,
        out_type=jax.ShapeDtypeStruct((_NC, np_, 16), jnp.float32),
        scratch_types=[
            pltpu.VMEM((tr, _CW), jnp.int32),
            pltpu.VMEM((_CW, 16), jnp.float32),
            pltpu.VMEM((_RSUB, 16), jnp.float32),
            pltpu.VMEM_SHARED((np_, 16), jnp.float32),
            pltpu.SemaphoreType.DMA,
        ],
    )
    def deg_kernel(dst2_hbm, out_hbm, idx2, ones_v, zbuf, acc, wsem):
        c = lax.axis_index("c")
        s = lax.axis_index("s")
        w = c * _NS + s

        pltpu.sync_copy(dst2_hbm.at[pl.ds(w * tr, tr)], idx2)

        def fill_ones(r, carry):
            ones_v[r, :] = jnp.ones((16,), jnp.float32)
            return carry

        lax.fori_loop(0, _CW, fill_ones, None)

        def fill_zero(r, carry):
            zbuf[r, :] = jnp.zeros((16,), jnp.float32)
            return carry

        lax.fori_loop(0, _RSUB, fill_zero, None)

        r0, nsub = _row_chunks(np_, s)

        def zero_acc(k, carry):
            pltpu.sync_copy(zbuf, acc.at[pl.ds(r0 + k * _RSUB, _RSUB)])
            return carry

        lax.fori_loop(0, nsub, zero_acc, None)
        plsc.subcore_barrier()

        def fire(j):
            pltpu.async_copy(ones_v, acc.at[idx2.at[j]], wsem, add=True)

        def drain_one():
            pltpu.make_async_copy(ones_v, acc.at[idx2.at[0]], wsem).wait()

        def prime(j, carry):
            fire(j)
            return carry

        lax.fori_loop(0, infl, prime, None)

        def steady(j, carry):
            drain_one()
            fire(j)
            return carry

        lax.fori_loop(infl, tr, steady, None)

        def drain(j, carry):
            drain_one()
            return carry

        lax.fori_loop(0, infl, drain, None)
        plsc.subcore_barrier()

        def wout(k, carry):
            rr = r0 + k * _RSUB
            pltpu.sync_copy(acc.at[pl.ds(rr, _RSUB)],
                            out_hbm.at[c, pl.ds(rr, _RSUB)])
            return carry

        lax.fori_loop(0, nsub, wout, None)

    return deg_kernel


@functools.lru_cache(maxsize=None)
def _make_agg(n, e, d):
    np_ = _NPAD
    crows = e // _CW
    tr = crows // (_NC * _NS)
    assert tr * _NC * _NS == crows and tr % 2 == 0 and tr % 8 == 0
    mesh = plsc.VectorSubcoreMesh(core_axis_name="c", subcore_axis_name="s")

    @functools.partial(
        pl.kernel,
        mesh=mesh,
        out_type=jax.ShapeDtypeStruct((_NC, np_, d), jnp.float32),
        scratch_types=[
            pltpu.VMEM((tr, _CW), jnp.int32),
            pltpu.VMEM((8, _CW), jnp.int32),
            pltpu.VMEM((_CW, d), jnp.float32),
            pltpu.VMEM((_CW, d), jnp.float32),
            pltpu.VMEM((_RSUB, d), jnp.float32),
            pltpu.VMEM_SHARED((np_, d), jnp.float32),
            pltpu.SemaphoreType.DMA,
            pltpu.SemaphoreType.DMA,
            pltpu.SemaphoreType.DMA,
            pltpu.SemaphoreType.DMA,
        ],
    )
    def agg_kernel(h_hbm, src2_hbm, dst2_hbm, out_hbm,
                   idx_s2, idd8, rows_a, rows_b,
                   zbuf, acc, gsem_a, gsem_b, wsem_a, wsem_b):
        c = lax.axis_index("c")
        s = lax.axis_index("s")
        w = c * _NS + s

        pltpu.sync_copy(src2_hbm.at[pl.ds(w * tr, tr)], idx_s2)

        def fill_zero(r, carry):
            def col(j, carry2):
                zbuf[r, pl.ds(j * 16, 16)] = jnp.zeros((16,), jnp.float32)
                return carry2
            return lax.fori_loop(0, d // 16, col, carry)

        lax.fori_loop(0, _RSUB, fill_zero, None)

        r0, nsub = _row_chunks(np_, s)

        def zero_acc(k, carry):
            pltpu.sync_copy(zbuf, acc.at[pl.ds(r0 + k * _RSUB, _RSUB)])
            return carry

        lax.fori_loop(0, nsub, zero_acc, None)
        plsc.subcore_barrier()

        def fire_a(j):
            pltpu.async_copy(h_hbm.at[idx_s2.at[j]], rows_a, gsem_a)

        def fire_b(j):
            pltpu.async_copy(h_hbm.at[idx_s2.at[j]], rows_b, gsem_b)

        def wait_ga():
            pltpu.make_async_copy(h_hbm.at[idx_s2.at[0]], rows_a,
                                  gsem_a).wait()

        def wait_gb():
            pltpu.make_async_copy(h_hbm.at[idx_s2.at[0]], rows_b,
                                  gsem_b).wait()

        def scatter(rows, k, sem):
            pltpu.async_copy(rows, acc.at[idd8.at[k]], sem, add=True)

        def wait_w(rows, sem):
            pltpu.make_async_copy(rows, acc.at[idd8.at[0]], sem).wait()

        def group(g, carry):
            j0 = 8 * g
            fire_a(j0)
            fire_b(j0 + 1)
            pltpu.sync_copy(dst2_hbm.at[pl.ds(w * tr + j0, 8)], idd8)

            def pair(k, carry2):
                wait_ga()
                scatter(rows_a, 2 * k, wsem_a)
                wait_gb()
                scatter(rows_b, 2 * k + 1, wsem_b)
                wait_w(rows_a, wsem_a)
                wait_w(rows_b, wsem_b)
                fire_a(j0 + 2 * k + 2)
                fire_b(j0 + 2 * k + 3)
                return carry2

            lax.fori_loop(0, 3, pair, None)
            wait_ga()
            scatter(rows_a, 6, wsem_a)
            wait_gb()
            scatter(rows_b, 7, wsem_b)
            wait_w(rows_a, wsem_a)
            wait_w(rows_b, wsem_b)
            return carry

        lax.fori_loop(0, tr // 8, group, None)
        plsc.subcore_barrier()

        def wout(k, carry):
            rr = r0 + k * _RSUB
            pltpu.sync_copy(acc.at[pl.ds(rr, _RSUB)],
                            out_hbm.at[c, pl.ds(rr, _RSUB)])
            return carry

        lax.fori_loop(0, nsub, wout, None)

    return agg_kernel


_BR = 1000


def _dinv_block(degp_ref):
    deg = degp_ref[0, :, 0:1] + degp_ref[1, :, 0:1] + 1.0
    return lax.rsqrt(deg)


def _tc1_body(x_ref, degp_ref, wp_ref, bp_ref, w1_ref, o_ref):
    dinv = _dinv_block(degp_ref)
    wc = jnp.dot(wp_ref[...], w1_ref[...], preferred_element_type=jnp.float32)
    bc = jnp.dot(bp_ref[...].reshape(1, -1), w1_ref[...],
                 preferred_element_type=jnp.float32)
    h = jnp.dot(x_ref[...], wc, preferred_element_type=jnp.float32) + bc
    o_ref[...] = h * dinv


def _tc2_body(a_ref, hs1_ref, degp_ref, b1_ref, w2_ref, o_ref):
    dinv = _dinv_block(degp_ref)
    t = (a_ref[0] + a_ref[1] + hs1_ref[...]) * dinv + b1_ref[...]
    t = jnp.maximum(t, 0.0)
    o_ref[...] = jnp.dot(t, w2_ref[...],
                         preferred_element_type=jnp.float32) * dinv


def _tc3_body(a_ref, hs2_ref, degp_ref, b2_ref, o_ref):
    dinv = _dinv_block(degp_ref)
    y = (a_ref[0] + a_ref[1] + hs2_ref[...]) * dinv + b2_ref[...]
    nrm = jnp.sqrt(jnp.sum(y * y, axis=1, keepdims=True))
    o_ref[...] = y / jnp.maximum(nrm, 1e-12)


def _row_spec(d):
    return pl.BlockSpec((_BR, d), lambda i: (i, 0))


def _part_spec(d):
    return pl.BlockSpec((_NC, _BR, d), lambda i: (0, i, 0))


def _full_spec(shape):
    nd = len(shape)
    return pl.BlockSpec(shape, lambda i: (0,) * nd)


@functools.lru_cache(maxsize=None)
def _make_tc1(n, d):
    return pl.pallas_call(
        _tc1_body,
        grid=(n // _BR,),
        in_specs=[_row_spec(d), _part_spec(16), _full_spec((d, d)),
                  _full_spec((d,)), _full_spec((d, d))],
        out_specs=_row_spec(d),
        out_shape=jax.ShapeDtypeStruct((n, d), jnp.float32),
    )


@functools.lru_cache(maxsize=None)
def _make_tc2(n, d):
    return pl.pallas_call(
        _tc2_body,
        grid=(n // _BR,),
        in_specs=[_part_spec(d), _row_spec(d), _part_spec(16),
                  _full_spec((d,)), _full_spec((d, d))],
        out_specs=_row_spec(d),
        out_shape=jax.ShapeDtypeStruct((n, d), jnp.float32),
    )


@functools.lru_cache(maxsize=None)
def _make_tc3(n, d):
    return pl.pallas_call(
        _tc3_body,
        grid=(n // _BR,),
        in_specs=[_part_spec(d), _row_spec(d), _part_spec(16),
                  _full_spec((d,))],
        out_specs=_row_spec(d),
        out_shape=jax.ShapeDtypeStruct((n, d), jnp.float32),
    )


def kernel(x, edge_index, W_pre, b_pre, W1, b1, W2, b2):
    n, d = x.shape
    e = edge_index.shape[1]
    ei = edge_index.astype(jnp.int32)
    src2 = ei[0].reshape(e // _CW, _CW)
    dst2 = ei[1].reshape(e // _CW, _CW)

    degp = _make_deg(n, e)(dst2)
    hs1 = _make_tc1(n, d)(x, degp, W_pre, b_pre, W1)
    agg1 = _make_agg(n, e, d)(hs1, src2, dst2)
    hs2 = _make_tc2(n, d)(agg1, hs1, degp, b1, W2)
    agg2 = _make_agg(n, e, d)(hs2, src2, dst2)
    return _make_tc3(n, d)(agg2, hs2, degp, b2)
